# Initial kernel scaffold; baseline (speedup 1.0000x reference)
import jax
import jax.numpy as jnp
from jax.experimental import pallas as pl


def kernel(x, edge_index, edge_weight, W1, W2):
    raise NotImplementedError("write your pallas kernel here")


# speedup vs baseline: 6.1884x; 6.1884x over previous
import functools

import jax
import jax.numpy as jnp
from jax import lax
from jax.experimental import pallas as pl
from jax.experimental.pallas import tpu as pltpu
from jax.experimental.pallas import tpu_sc as plsc

_N = 10000
_E = 320000
_D = 128
_H = 64
_C = 16

_NT = 16
_NW = 32
_K = 80
_NCHUNK = _E // (_NW * _K)
_ZROWS = 125


def _make_spmm(hdim):
    mesh = plsc.VectorSubcoreMesh(core_axis_name="c", subcore_axis_name="s")

    @functools.partial(
        pl.kernel,
        mesh=mesh,
        compiler_params=pltpu.CompilerParams(use_tc_tiling_on_sc=False),
        out_type=jax.ShapeDtypeStruct((2, _N, hdim), jnp.float32),
        scratch_types=[
            pltpu.VMEM((_NCHUNK, _K), jnp.int32),
            pltpu.VMEM((_NCHUNK, _K), jnp.int32),
            pltpu.VMEM((_NCHUNK, _K), jnp.float32),
            pltpu.VMEM((_K, hdim), jnp.float32),
            pltpu.VMEM((_ZROWS, hdim), jnp.float32),
            pltpu.VMEM_SHARED((_N, hdim), jnp.float32),
            pltpu.SemaphoreType.DMA,
        ],
    )
    def spmm(m_hbm, src_hbm, dst_hbm, ew_hbm, out_hbm,
             src_v, dst_v, ew_v, rows_v, zero_v, acc_sh, sem):
        cid = lax.axis_index("c")
        sid = lax.axis_index("s")
        wid = cid * _NT + sid

        pltpu.sync_copy(src_hbm.at[wid], src_v)
        pltpu.sync_copy(dst_hbm.at[wid], dst_v)
        pltpu.sync_copy(ew_hbm.at[wid], ew_v)

        def zero_row(r, carry):
            for j in range(hdim // 16):
                zero_v[r, pl.ds(j * 16, 16)] = jnp.zeros((16,), jnp.float32)
            return carry
        lax.fori_loop(0, _ZROWS, zero_row, 0)

        def zero_copy(q, carry):
            pltpu.sync_copy(
                zero_v,
                acc_sh.at[pl.ds(sid * (_N // _NT) + q * _ZROWS, _ZROWS)])
            return carry
        lax.fori_loop(0, (_N // _NT) // _ZROWS, zero_copy, 0)
        plsc.subcore_barrier()

        def chunk(g, carry):
            pltpu.async_copy(m_hbm.at[src_v.at[g]], rows_v, sem).wait()

            def egroup(eg, c2):
                base = eg * 16
                ew16 = ew_v[g, pl.ds(base, 16)]
                for i in range(16):
                    wv = lax.broadcast(ew16[i], (16,))
                    for j in range(hdim // 16):
                        rows_v[base + i, pl.ds(j * 16, 16)] = (
                            rows_v[base + i, pl.ds(j * 16, 16)] * wv)
                return c2
            lax.fori_loop(0, _K // 16, egroup, 0)

            pltpu.sync_copy(rows_v, acc_sh.at[dst_v.at[g]], add=True)
            return carry
        lax.fori_loop(0, _NCHUNK, chunk, 0)
        plsc.subcore_barrier()

        @pl.when(sid < 10)
        def _():
            pltpu.sync_copy(acc_sh.at[pl.ds(sid * 1000, 1000)],
                            out_hbm.at[cid, pl.ds(sid * 1000, 1000)])

    return spmm


_spmm_h = _make_spmm(_H)
_spmm_c = _make_spmm(_C)


def _mm_body(x_ref, w_ref, o_ref):
    o_ref[...] = jnp.dot(x_ref[...], w_ref[...],
                         preferred_element_type=jnp.float32)


def _fuse_body(p0_ref, p1_ref, w_ref, o_ref):
    h = jnp.maximum(p0_ref[...] + p1_ref[...], 0.0)
    o_ref[...] = jnp.dot(h, w_ref[...], preferred_element_type=jnp.float32)


def _add_body(a_ref, b_ref, o_ref):
    o_ref[...] = a_ref[...] + b_ref[...]


_RB = 2000


def _mm1(x, w):
    return pl.pallas_call(
        _mm_body,
        grid=(_N // _RB,),
        in_specs=[
            pl.BlockSpec((_RB, _D), lambda i: (i, 0)),
            pl.BlockSpec((_D, _H), lambda i: (0, 0)),
        ],
        out_specs=pl.BlockSpec((_RB, _H), lambda i: (i, 0)),
        out_shape=jax.ShapeDtypeStruct((_N, _H), jnp.float32),
    )(x, w)


def _fuse2(p0, p1, w):
    return pl.pallas_call(
        _fuse_body,
        grid=(_N // _RB,),
        in_specs=[
            pl.BlockSpec((_RB, _H), lambda i: (i, 0)),
            pl.BlockSpec((_RB, _H), lambda i: (i, 0)),
            pl.BlockSpec((_H, _C), lambda i: (0, 0)),
        ],
        out_specs=pl.BlockSpec((_RB, _C), lambda i: (i, 0)),
        out_shape=jax.ShapeDtypeStruct((_N, _C), jnp.float32),
    )(p0, p1, w)


def _final_add(a, b):
    return pl.pallas_call(
        _add_body,
        out_shape=jax.ShapeDtypeStruct((_N, _C), jnp.float32),
    )(a, b)


def kernel(x, edge_index, edge_weight, W1, W2):
    src = edge_index[0].reshape(_NW, _NCHUNK, _K)
    dst = edge_index[1].reshape(_NW, _NCHUNK, _K)
    ew = edge_weight.reshape(_NW, _NCHUNK, _K)

    m1 = _mm1(x, W1)
    p = _spmm_h(m1, src, dst, ew)
    m2 = _fuse2(p[0], p[1], W2)
    q = _spmm_c(m2, src, dst, ew)
    return _final_add(q[0], q[1])


# speedup vs baseline: 7.0148x; 1.1335x over previous
import functools

import jax
import jax.numpy as jnp
from jax import lax
from jax.experimental import pallas as pl
from jax.experimental.pallas import tpu as pltpu
from jax.experimental.pallas import tpu_sc as plsc

_N = 10000
_E = 320000
_D = 128
_H = 64
_C = 16

_NT = 16
_NW = 32
_K = 80
_NCHUNK = 128
_EPT = _NCHUNK * _K
_EPAD = _NW * _EPT
_NBUF = 4
_PF = 2
_ZROWS = 125


def _make_spmm(hdim):
    mesh = plsc.VectorSubcoreMesh(core_axis_name="c", subcore_axis_name="s")

    @functools.partial(
        pl.kernel,
        mesh=mesh,
        compiler_params=pltpu.CompilerParams(use_tc_tiling_on_sc=False),
        out_type=jax.ShapeDtypeStruct((2, _N, hdim), jnp.float32),
        scratch_types=[
            pltpu.VMEM((_NCHUNK, _K), jnp.int32),
            pltpu.VMEM((_NCHUNK, _K), jnp.int32),
            pltpu.VMEM((_EPT,), jnp.float32),
            pltpu.VMEM((_ZROWS, hdim), jnp.float32),
            pltpu.VMEM_SHARED((_N, hdim), jnp.float32),
        ] + [pltpu.VMEM((_K, hdim), jnp.float32) for _ in range(_NBUF)]
          + [pltpu.SemaphoreType.DMA for _ in range(2 * _NBUF)],
    )
    def spmm(m_hbm, src_hbm, dst_hbm, ew_hbm, out_hbm,
             src_v, dst_v, ew_v, zero_v, acc_sh,
             rb0, rb1, rb2, rb3,
             gs0, gs1, gs2, gs3, ss0, ss1, ss2, ss3):
        rbufs = (rb0, rb1, rb2, rb3)
        gsems = (gs0, gs1, gs2, gs3)
        ssems = (ss0, ss1, ss2, ss3)

        cid = lax.axis_index("c")
        sid = lax.axis_index("s")
        wid = cid * _NT + sid

        pltpu.sync_copy(src_hbm.at[wid], src_v)
        pltpu.sync_copy(dst_hbm.at[wid], dst_v)
        pltpu.sync_copy(ew_hbm.at[wid], ew_v)

        def zero_row(r, carry):
            for j in range(hdim // 16):
                zero_v[r, pl.ds(j * 16, 16)] = jnp.zeros((16,), jnp.float32)
            return carry
        lax.fori_loop(0, _ZROWS, zero_row, 0)

        def zero_copy(q, carry):
            pltpu.sync_copy(
                zero_v,
                acc_sh.at[pl.ds(sid * (_N // _NT) + q * _ZROWS, _ZROWS)])
            return carry
        lax.fori_loop(0, (_N // _NT) // _ZROWS, zero_copy, 0)
        plsc.subcore_barrier()

        def start_gather(b, g):
            pltpu.async_copy(m_hbm.at[src_v.at[g]], rbufs[b], gsems[b])

        def wait_gather(b):
            pltpu.make_async_copy(m_hbm.at[src_v.at[0]], rbufs[b],
                                  gsems[b]).wait()

        def start_scatter(b, g):
            pltpu.async_copy(rbufs[b], acc_sh.at[dst_v.at[g]], ssems[b],
                             add=True)

        def wait_scatter(b):
            pltpu.make_async_copy(rbufs[b], acc_sh.at[dst_v.at[0]],
                                  ssems[b]).wait()

        def scale(b, g):
            def grp(eg, carry):
                ew16 = ew_v[pl.ds(g * _K + eg * 16, 16)]
                base = eg * 16
                for i in range(16):
                    wv = lax.broadcast(ew16[i], (16,))
                    for j in range(hdim // 16):
                        rbufs[b][base + i, pl.ds(j * 16, 16)] = (
                            rbufs[b][base + i, pl.ds(j * 16, 16)] * wv)
                return carry
            lax.fori_loop(0, _K // 16, grp, 0)

        def step(b, g, pre_b, pre_g, do_wait_scatter, do_prefetch):
            if do_prefetch:
                if do_wait_scatter:
                    wait_scatter(pre_b)
                start_gather(pre_b, pre_g)
            wait_gather(b)
            scale(b, g)
            start_scatter(b, g)

        start_gather(0, 0)
        start_gather(1, 1)
        for b in range(_NBUF):
            g = b
            step(b, g, (b + _PF) % _NBUF, g + _PF,
                 do_wait_scatter=(b >= _PF), do_prefetch=True)

        def body4(i, carry):
            for b in range(_NBUF):
                g = i * _NBUF + b
                step(b, g, (b + _PF) % _NBUF, g + _PF,
                     do_wait_scatter=True, do_prefetch=True)
            return carry
        lax.fori_loop(1, _NCHUNK // _NBUF - 1, body4, 0)

        for b in range(_NBUF):
            g = _NCHUNK - _NBUF + b
            pf = b < _NBUF - _PF
            step(b, g, (b + _PF) % _NBUF, g + _PF,
                 do_wait_scatter=pf, do_prefetch=pf)
        for b in range(_NBUF):
            wait_scatter(b)
        plsc.subcore_barrier()

        @pl.when(sid < 10)
        def _():
            pltpu.sync_copy(acc_sh.at[pl.ds(sid * 1000, 1000)],
                            out_hbm.at[cid, pl.ds(sid * 1000, 1000)])

    return spmm


_spmm_h = _make_spmm(_H)
_spmm_c = _make_spmm(_C)


def _mm_body(x_ref, w_ref, o_ref):
    o_ref[...] = jnp.dot(x_ref[...], w_ref[...],
                         preferred_element_type=jnp.float32)


def _fuse_body(p0_ref, p1_ref, w_ref, o_ref):
    h = jnp.maximum(p0_ref[...] + p1_ref[...], 0.0)
    o_ref[...] = jnp.dot(h, w_ref[...], preferred_element_type=jnp.float32)


def _add_body(a_ref, b_ref, o_ref):
    o_ref[...] = a_ref[...] + b_ref[...]


_RB = 2000


def _mm1(x, w):
    return pl.pallas_call(
        _mm_body,
        grid=(_N // _RB,),
        in_specs=[
            pl.BlockSpec((_RB, _D), lambda i: (i, 0)),
            pl.BlockSpec((_D, _H), lambda i: (0, 0)),
        ],
        out_specs=pl.BlockSpec((_RB, _H), lambda i: (i, 0)),
        out_shape=jax.ShapeDtypeStruct((_N, _H), jnp.float32),
    )(x, w)


def _fuse2(p0, p1, w):
    return pl.pallas_call(
        _fuse_body,
        grid=(_N // _RB,),
        in_specs=[
            pl.BlockSpec((_RB, _H), lambda i: (i, 0)),
            pl.BlockSpec((_RB, _H), lambda i: (i, 0)),
            pl.BlockSpec((_H, _C), lambda i: (0, 0)),
        ],
        out_specs=pl.BlockSpec((_RB, _C), lambda i: (i, 0)),
        out_shape=jax.ShapeDtypeStruct((_N, _C), jnp.float32),
    )(p0, p1, w)


def _final_add(a, b):
    return pl.pallas_call(
        _add_body,
        out_shape=jax.ShapeDtypeStruct((_N, _C), jnp.float32),
    )(a, b)


def kernel(x, edge_index, edge_weight, W1, W2):
    pad = _EPAD - _E
    src = jnp.concatenate(
        [edge_index[0], jnp.zeros((pad,), jnp.int32)]).reshape(
            _NW, _NCHUNK, _K)
    dst = jnp.concatenate(
        [edge_index[1], jnp.zeros((pad,), jnp.int32)]).reshape(
            _NW, _NCHUNK, _K)
    ew = jnp.concatenate(
        [edge_weight, jnp.zeros((pad,), jnp.float32)]).reshape(_NW, _EPT)

    m1 = _mm1(x, W1)
    p = _spmm_h(m1, src, dst, ew)
    m2 = _fuse2(p[0], p[1], W2)
    q = _spmm_c(m2, src, dst, ew)
    return _final_add(q[0], q[1])


# speedup vs baseline: 10.1112x; 1.4414x over previous
import functools

import jax
import jax.numpy as jnp
from jax import lax
from jax.experimental import pallas as pl
from jax.experimental.pallas import tpu as pltpu
from jax.experimental.pallas import tpu_sc as plsc

_N = 10000
_E = 320000
_D = 128
_H = 64
_C = 16

_NT = 16
_NW = 32
_K = 80
_NCHUNK = 125
_EPT = _NCHUNK * _K
_NBUF = 4
_PF = 2
_ZROWS = 125


def _make_spmm(hdim):
    mesh = plsc.VectorSubcoreMesh(core_axis_name="c", subcore_axis_name="s")

    @functools.partial(
        pl.kernel,
        mesh=mesh,
        compiler_params=pltpu.CompilerParams(use_tc_tiling_on_sc=False),
        out_type=jax.ShapeDtypeStruct((2, _N, hdim), jnp.float32),
        scratch_types=[
            pltpu.VMEM((_NCHUNK, _K), jnp.int32),
            pltpu.VMEM((_NCHUNK, _K), jnp.int32),
            pltpu.VMEM((_EPT,), jnp.float32),
            pltpu.VMEM((_ZROWS, hdim), jnp.float32),
            pltpu.VMEM_SHARED((_N, hdim), jnp.float32),
        ] + [pltpu.VMEM((_K, hdim), jnp.float32) for _ in range(_NBUF)]
          + [pltpu.SemaphoreType.DMA for _ in range(2 * _NBUF)],
    )
    def spmm(m_hbm, src_hbm, dst_hbm, ew_hbm, out_hbm,
             src_v, dst_v, ew_v, zero_v, acc_sh,
             rb0, rb1, rb2, rb3,
             gs0, gs1, gs2, gs3, ss0, ss1, ss2, ss3):
        rbufs = (rb0, rb1, rb2, rb3)
        gsems = (gs0, gs1, gs2, gs3)
        ssems = (ss0, ss1, ss2, ss3)

        cid = lax.axis_index("c")
        sid = lax.axis_index("s")
        wid = cid * _NT + sid

        pltpu.sync_copy(src_hbm.at[wid], src_v)
        pltpu.sync_copy(dst_hbm.at[wid], dst_v)
        pltpu.sync_copy(ew_hbm.at[wid], ew_v)

        def zero_row(r, carry):
            for j in range(hdim // 16):
                zero_v[r, pl.ds(j * 16, 16)] = jnp.zeros((16,), jnp.float32)
            return carry
        lax.fori_loop(0, _ZROWS, zero_row, 0)

        def zero_copy(q, carry):
            pltpu.sync_copy(
                zero_v,
                acc_sh.at[pl.ds(sid * (_N // _NT) + q * _ZROWS, _ZROWS)])
            return carry
        lax.fori_loop(0, (_N // _NT) // _ZROWS, zero_copy, 0)
        plsc.subcore_barrier()

        def start_gather(b, g):
            pltpu.async_copy(m_hbm.at[src_v.at[g]], rbufs[b], gsems[b])

        def wait_gather(b):
            pltpu.make_async_copy(m_hbm.at[src_v.at[0]], rbufs[b],
                                  gsems[b]).wait()

        def start_scatter(b, g):
            pltpu.async_copy(rbufs[b], acc_sh.at[dst_v.at[g]], ssems[b],
                             add=True)

        def wait_scatter(b):
            pltpu.make_async_copy(rbufs[b], acc_sh.at[dst_v.at[0]],
                                  ssems[b]).wait()

        def scale(b, g):
            def grp(eg, carry):
                ew16 = ew_v[pl.ds(g * _K + eg * 16, 16)]
                base = eg * 16
                for i in range(16):
                    wv = lax.broadcast(ew16[i], (16,))
                    for j in range(hdim // 16):
                        rbufs[b][base + i, pl.ds(j * 16, 16)] = (
                            rbufs[b][base + i, pl.ds(j * 16, 16)] * wv)
                return carry
            lax.fori_loop(0, _K // 16, grp, 0)

        def step(b, g, pre_b, pre_g, do_wait_scatter, do_prefetch):
            if do_prefetch:
                if do_wait_scatter:
                    wait_scatter(pre_b)
                start_gather(pre_b, pre_g)
            wait_gather(b)
            scale(b, g)
            start_scatter(b, g)

        start_gather(0, 0)
        start_gather(1, 1)
        for b in range(_NBUF):
            g = b
            step(b, g, (b + _PF) % _NBUF, g + _PF,
                 do_wait_scatter=(b >= _PF), do_prefetch=True)

        def body4(i, carry):
            for b in range(_NBUF):
                g = i * _NBUF + b
                step(b, g, (b + _PF) % _NBUF, g + _PF,
                     do_wait_scatter=True, do_prefetch=True)
            return carry
        lax.fori_loop(1, 30, body4, 0)

        for b in range(_NBUF):
            g = 120 + b
            pf = g + _PF < _NCHUNK
            step(b, g, (b + _PF) % _NBUF, g + _PF,
                 do_wait_scatter=pf, do_prefetch=pf)
        step(0, 124, 0, 0, do_wait_scatter=False, do_prefetch=False)
        for b in range(_NBUF):
            wait_scatter(b)
        plsc.subcore_barrier()

        @pl.when(sid < 10)
        def _():
            pltpu.sync_copy(acc_sh.at[pl.ds(sid * 1000, 1000)],
                            out_hbm.at[cid, pl.ds(sid * 1000, 1000)])

    return spmm


_spmm_h = _make_spmm(_H)
_spmm_c = _make_spmm(_C)


def _mm_body(x_ref, w_ref, o_ref):
    o_ref[...] = jnp.dot(x_ref[...], w_ref[...],
                         preferred_element_type=jnp.float32)


def _fuse_body(p0_ref, p1_ref, w_ref, o_ref):
    h = jnp.maximum(p0_ref[...] + p1_ref[...], 0.0)
    o_ref[...] = jnp.dot(h, w_ref[...], preferred_element_type=jnp.float32)


def _add_body(a_ref, b_ref, o_ref):
    o_ref[...] = a_ref[...] + b_ref[...]


_RB = 2000


def _mm1(x, w):
    return pl.pallas_call(
        _mm_body,
        grid=(_N // _RB,),
        in_specs=[
            pl.BlockSpec((_RB, _D), lambda i: (i, 0)),
            pl.BlockSpec((_D, _H), lambda i: (0, 0)),
        ],
        out_specs=pl.BlockSpec((_RB, _H), lambda i: (i, 0)),
        out_shape=jax.ShapeDtypeStruct((_N, _H), jnp.float32),
    )(x, w)


def _fuse2(p0, p1, w):
    return pl.pallas_call(
        _fuse_body,
        grid=(_N // _RB,),
        in_specs=[
            pl.BlockSpec((_RB, _H), lambda i: (i, 0)),
            pl.BlockSpec((_RB, _H), lambda i: (i, 0)),
            pl.BlockSpec((_H, _C), lambda i: (0, 0)),
        ],
        out_specs=pl.BlockSpec((_RB, _C), lambda i: (i, 0)),
        out_shape=jax.ShapeDtypeStruct((_N, _C), jnp.float32),
    )(p0, p1, w)


def _final_add(a, b):
    return pl.pallas_call(
        _add_body,
        out_shape=jax.ShapeDtypeStruct((_N, _C), jnp.float32),
    )(a, b)


def kernel(x, edge_index, edge_weight, W1, W2):
    src = edge_index[0].reshape(_NW, _NCHUNK, _K)
    dst = edge_index[1].reshape(_NW, _NCHUNK, _K)
    ew = edge_weight.reshape(_NW, _EPT)

    m1 = _mm1(x, W1)
    p = _spmm_h(m1, src, dst, ew)
    m2 = _fuse2(p[0], p[1], W2)
    q = _spmm_c(m2, src, dst, ew)
    return _final_add(q[0], q[1])


# speedup vs baseline: 10.9166x; 1.0797x over previous
import functools

import jax
import jax.numpy as jnp
from jax import lax
from jax.experimental import pallas as pl
from jax.experimental.pallas import tpu as pltpu
from jax.experimental.pallas import tpu_sc as plsc

_N = 10000
_E = 320000
_D = 128
_H = 64
_C = 16

_NT = 16
_NW = 32
_K = 80
_NCHUNK = 125
_EPT = _NCHUNK * _K
_NBUF = 4
_PF = 2
_ZROWS = 125


def _make_spmm(hdim):
    mesh = plsc.VectorSubcoreMesh(core_axis_name="c", subcore_axis_name="s")

    @functools.partial(
        pl.kernel,
        mesh=mesh,
        compiler_params=pltpu.CompilerParams(use_tc_tiling_on_sc=False),
        out_type=jax.ShapeDtypeStruct((2, _N, hdim), jnp.float32),
        scratch_types=[
            pltpu.VMEM((_NCHUNK, _K), jnp.int32),
            pltpu.VMEM((_NCHUNK, _K), jnp.int32),
            pltpu.VMEM((_EPT,), jnp.float32),
            pltpu.VMEM((_ZROWS, hdim), jnp.float32),
            pltpu.VMEM_SHARED((_N, hdim), jnp.float32),
        ] + [pltpu.VMEM((_K, hdim), jnp.float32) for _ in range(_NBUF)]
          + [pltpu.SemaphoreType.DMA for _ in range(2 * _NBUF)],
    )
    def spmm(m_hbm, edge_hbm, ew_hbm, out_hbm,
             src_v, dst_v, ew_v, zero_v, acc_sh,
             rb0, rb1, rb2, rb3,
             gs0, gs1, gs2, gs3, ss0, ss1, ss2, ss3):
        rbufs = (rb0, rb1, rb2, rb3)
        gsems = (gs0, gs1, gs2, gs3)
        ssems = (ss0, ss1, ss2, ss3)

        cid = lax.axis_index("c")
        sid = lax.axis_index("s")
        wid = cid * _NT + sid

        pltpu.sync_copy(edge_hbm.at[0, wid], src_v)
        pltpu.sync_copy(edge_hbm.at[1, wid], dst_v)
        pltpu.sync_copy(ew_hbm.at[wid], ew_v)

        def zero_row(r, carry):
            for j in range(hdim // 16):
                zero_v[r, pl.ds(j * 16, 16)] = jnp.zeros((16,), jnp.float32)
            return carry
        lax.fori_loop(0, _ZROWS, zero_row, 0)

        def zero_copy(q, carry):
            pltpu.sync_copy(
                zero_v,
                acc_sh.at[pl.ds(sid * (_N // _NT) + q * _ZROWS, _ZROWS)])
            return carry
        lax.fori_loop(0, (_N // _NT) // _ZROWS, zero_copy, 0)
        plsc.subcore_barrier()

        def start_gather(b, g):
            pltpu.async_copy(m_hbm.at[src_v.at[g]], rbufs[b], gsems[b])

        def wait_gather(b):
            pltpu.make_async_copy(m_hbm.at[src_v.at[0]], rbufs[b],
                                  gsems[b]).wait()

        def start_scatter(b, g):
            pltpu.async_copy(rbufs[b], acc_sh.at[dst_v.at[g]], ssems[b],
                             add=True)

        def wait_scatter(b):
            pltpu.make_async_copy(rbufs[b], acc_sh.at[dst_v.at[0]],
                                  ssems[b]).wait()

        def scale(b, g):
            def grp(eg, carry):
                ew16 = ew_v[pl.ds(g * _K + eg * 16, 16)]
                base = eg * 16
                for i in range(16):
                    wv = lax.broadcast(ew16[i], (16,))
                    for j in range(hdim // 16):
                        rbufs[b][base + i, pl.ds(j * 16, 16)] = (
                            rbufs[b][base + i, pl.ds(j * 16, 16)] * wv)
                return carry
            lax.fori_loop(0, _K // 16, grp, 0)

        def step(b, g, pre_b, pre_g, do_wait_scatter, do_prefetch):
            if do_prefetch:
                if do_wait_scatter:
                    wait_scatter(pre_b)
                start_gather(pre_b, pre_g)
            wait_gather(b)
            scale(b, g)
            start_scatter(b, g)

        start_gather(0, 0)
        start_gather(1, 1)
        for b in range(_NBUF):
            g = b
            step(b, g, (b + _PF) % _NBUF, g + _PF,
                 do_wait_scatter=(b >= _PF), do_prefetch=True)

        def body4(i, carry):
            for b in range(_NBUF):
                g = i * _NBUF + b
                step(b, g, (b + _PF) % _NBUF, g + _PF,
                     do_wait_scatter=True, do_prefetch=True)
            return carry
        lax.fori_loop(1, 30, body4, 0)

        for b in range(_NBUF):
            g = 120 + b
            pf = g + _PF < _NCHUNK
            step(b, g, (b + _PF) % _NBUF, g + _PF,
                 do_wait_scatter=pf, do_prefetch=pf)
        step(0, 124, 0, 0, do_wait_scatter=False, do_prefetch=False)
        for b in range(_NBUF):
            wait_scatter(b)
        plsc.subcore_barrier()

        @pl.when(sid < 10)
        def _():
            pltpu.sync_copy(acc_sh.at[pl.ds(sid * 1000, 1000)],
                            out_hbm.at[cid, pl.ds(sid * 1000, 1000)])

    return spmm


_spmm_h = _make_spmm(_H)
_spmm_c = _make_spmm(_C)


def _mm_body(x_ref, w_ref, o_ref):
    o_ref[...] = jnp.dot(x_ref[...], w_ref[...],
                         preferred_element_type=jnp.float32)


def _fuse_body(p0_ref, p1_ref, w_ref, o_ref):
    h = jnp.maximum(p0_ref[0] + p1_ref[0], 0.0)
    o_ref[...] = jnp.dot(h, w_ref[...], preferred_element_type=jnp.float32)


def _add_body(a_ref, b_ref, o_ref):
    o_ref[...] = a_ref[0] + b_ref[0]


_RB = 2000


def _mm1(x, w):
    return pl.pallas_call(
        _mm_body,
        grid=(_N // _RB,),
        in_specs=[
            pl.BlockSpec((_RB, _D), lambda i: (i, 0)),
            pl.BlockSpec((_D, _H), lambda i: (0, 0)),
        ],
        out_specs=pl.BlockSpec((_RB, _H), lambda i: (i, 0)),
        out_shape=jax.ShapeDtypeStruct((_N, _H), jnp.float32),
    )(x, w)


def _fuse2(p, w):
    return pl.pallas_call(
        _fuse_body,
        grid=(_N // _RB,),
        in_specs=[
            pl.BlockSpec((1, _RB, _H), lambda i: (0, i, 0)),
            pl.BlockSpec((1, _RB, _H), lambda i: (1, i, 0)),
            pl.BlockSpec((_H, _C), lambda i: (0, 0)),
        ],
        out_specs=pl.BlockSpec((_RB, _C), lambda i: (i, 0)),
        out_shape=jax.ShapeDtypeStruct((_N, _C), jnp.float32),
    )(p, p, w)


def _final_add(q):
    return pl.pallas_call(
        _add_body,
        grid=(1,),
        in_specs=[
            pl.BlockSpec((1, _N, _C), lambda i: (0, 0, 0)),
            pl.BlockSpec((1, _N, _C), lambda i: (1, 0, 0)),
        ],
        out_specs=pl.BlockSpec((_N, _C), lambda i: (0, 0)),
        out_shape=jax.ShapeDtypeStruct((_N, _C), jnp.float32),
    )(q, q)


def kernel(x, edge_index, edge_weight, W1, W2):
    edges = edge_index.reshape(2, _NW, _NCHUNK, _K)
    ew = edge_weight.reshape(_NW, _EPT)

    m1 = _mm1(x, W1)
    p = _spmm_h(m1, edges, ew)
    m2 = _fuse2(p, W2)
    q = _spmm_c(m2, edges, ew)
    return _final_add(q)


# speedup vs baseline: 19.3772x; 1.7750x over previous
import functools

import jax
import jax.numpy as jnp
from jax import lax
from jax.experimental import pallas as pl
from jax.experimental.pallas import tpu as pltpu
from jax.experimental.pallas import tpu_sc as plsc

_N = 10000
_E = 320000
_D = 128
_H = 64
_C = 16

_NT = 16
_NW = 32
_K = 80
_NCHUNK = 125
_EPT = _NCHUNK * _K
_NBUF = 4
_PF = 2
_ZROWS = 125


def _make_spmm(hdim, dtype):
    mesh = plsc.VectorSubcoreMesh(core_axis_name="c", subcore_axis_name="s")
    lanes = 32 if dtype == jnp.bfloat16 else 16

    @functools.partial(
        pl.kernel,
        mesh=mesh,
        compiler_params=pltpu.CompilerParams(use_tc_tiling_on_sc=False,
                                             needs_layout_passes=False),
        out_type=jax.ShapeDtypeStruct((2, _N, hdim), dtype),
        scratch_types=[
            pltpu.VMEM((_NCHUNK, _K), jnp.int32),
            pltpu.VMEM((_NCHUNK, _K), jnp.int32),
            pltpu.VMEM((_EPT,), jnp.float32),
            pltpu.VMEM((_ZROWS, hdim), dtype),
            pltpu.VMEM_SHARED((_N, hdim), dtype),
        ] + [pltpu.VMEM((_K, hdim), dtype) for _ in range(_NBUF)]
          + [pltpu.SemaphoreType.DMA for _ in range(2 * _NBUF)],
    )
    def spmm(m_hbm, edge_hbm, ew_hbm, out_hbm,
             src_v, dst_v, ew_v, zero_v, acc_sh,
             rb0, rb1, rb2, rb3,
             gs0, gs1, gs2, gs3, ss0, ss1, ss2, ss3):
        rbufs = (rb0, rb1, rb2, rb3)
        gsems = (gs0, gs1, gs2, gs3)
        ssems = (ss0, ss1, ss2, ss3)

        cid = lax.axis_index("c")
        sid = lax.axis_index("s")
        wid = cid * _NT + sid

        pltpu.sync_copy(edge_hbm.at[0, wid], src_v)
        pltpu.sync_copy(edge_hbm.at[1, wid], dst_v)
        pltpu.sync_copy(ew_hbm.at[wid], ew_v)

        def zero_row(r, carry):
            for j in range(hdim // lanes):
                zero_v[r, pl.ds(j * lanes, lanes)] = jnp.zeros((lanes,), dtype)
            return carry
        lax.fori_loop(0, _ZROWS, zero_row, 0)

        def zero_copy(q, carry):
            pltpu.sync_copy(
                zero_v,
                acc_sh.at[pl.ds(sid * (_N // _NT) + q * _ZROWS, _ZROWS)])
            return carry
        lax.fori_loop(0, (_N // _NT) // _ZROWS, zero_copy, 0)
        plsc.subcore_barrier()

        def start_gather(b, g):
            pltpu.async_copy(m_hbm.at[src_v.at[g]], rbufs[b], gsems[b])

        def wait_gather(b):
            pltpu.make_async_copy(m_hbm.at[src_v.at[0]], rbufs[b],
                                  gsems[b]).wait()

        def start_scatter(b, g):
            pltpu.async_copy(rbufs[b], acc_sh.at[dst_v.at[g]], ssems[b],
                             add=True)

        def wait_scatter(b):
            pltpu.make_async_copy(rbufs[b], acc_sh.at[dst_v.at[0]],
                                  ssems[b]).wait()

        def scale(b, g):
            def grp(eg, carry):
                ew16 = ew_v[pl.ds(g * _K + eg * 16, 16)]
                base = eg * 16
                for i in range(16):
                    wv = lax.broadcast(ew16[i], (16,))
                    if dtype == jnp.bfloat16:
                        wv = plsc.pack(wv, wv,
                                       format=plsc.PackFormat.INTERLEAVED)
                    for j in range(hdim // lanes):
                        rbufs[b][base + i, pl.ds(j * lanes, lanes)] = (
                            rbufs[b][base + i, pl.ds(j * lanes, lanes)] * wv)
                return carry
            lax.fori_loop(0, _K // 16, grp, 0)

        def step(b, g, pre_b, pre_g, do_wait_scatter, do_prefetch):
            if do_prefetch:
                if do_wait_scatter:
                    wait_scatter(pre_b)
                start_gather(pre_b, pre_g)
            wait_gather(b)
            scale(b, g)
            start_scatter(b, g)

        start_gather(0, 0)
        start_gather(1, 1)
        for b in range(_NBUF):
            g = b
            step(b, g, (b + _PF) % _NBUF, g + _PF,
                 do_wait_scatter=(b >= _PF), do_prefetch=True)

        def body4(i, carry):
            for b in range(_NBUF):
                g = i * _NBUF + b
                step(b, g, (b + _PF) % _NBUF, g + _PF,
                     do_wait_scatter=True, do_prefetch=True)
            return carry
        lax.fori_loop(1, 30, body4, 0)

        for b in range(_NBUF):
            g = 120 + b
            pf = g + _PF < _NCHUNK
            step(b, g, (b + _PF) % _NBUF, g + _PF,
                 do_wait_scatter=pf, do_prefetch=pf)
        step(0, 124, 0, 0, do_wait_scatter=False, do_prefetch=False)
        for b in range(_NBUF):
            wait_scatter(b)
        plsc.subcore_barrier()

        @pl.when(sid < 10)
        def _():
            pltpu.sync_copy(acc_sh.at[pl.ds(sid * 1000, 1000)],
                            out_hbm.at[cid, pl.ds(sid * 1000, 1000)])

    return spmm


_spmm_h = _make_spmm(_H, jnp.bfloat16)
_spmm_c = _make_spmm(_C, jnp.float32)


def _mm_body(x_ref, w_ref, o_ref):
    o_ref[...] = jnp.dot(x_ref[...], w_ref[...],
                         preferred_element_type=jnp.float32
                         ).astype(jnp.bfloat16)


def _fuse_body(p0_ref, p1_ref, w_ref, o_ref):
    h = jnp.maximum(p0_ref[0].astype(jnp.float32)
                    + p1_ref[0].astype(jnp.float32), 0.0)
    o_ref[...] = jnp.dot(h, w_ref[...], preferred_element_type=jnp.float32)


def _add_body(a_ref, b_ref, o_ref):
    o_ref[...] = a_ref[0] + b_ref[0]


_RB = 2000


def _mm1(x, w):
    return pl.pallas_call(
        _mm_body,
        grid=(_N // _RB,),
        in_specs=[
            pl.BlockSpec((_RB, _D), lambda i: (i, 0)),
            pl.BlockSpec((_D, _H), lambda i: (0, 0)),
        ],
        out_specs=pl.BlockSpec((_RB, _H), lambda i: (i, 0)),
        out_shape=jax.ShapeDtypeStruct((_N, _H), jnp.bfloat16),
    )(x, w)


def _fuse2(p, w):
    return pl.pallas_call(
        _fuse_body,
        grid=(_N // _RB,),
        in_specs=[
            pl.BlockSpec((1, _RB, _H), lambda i: (0, i, 0)),
            pl.BlockSpec((1, _RB, _H), lambda i: (1, i, 0)),
            pl.BlockSpec((_H, _C), lambda i: (0, 0)),
        ],
        out_specs=pl.BlockSpec((_RB, _C), lambda i: (i, 0)),
        out_shape=jax.ShapeDtypeStruct((_N, _C), jnp.float32),
    )(p, p, w)


def _final_add(q):
    return pl.pallas_call(
        _add_body,
        grid=(1,),
        in_specs=[
            pl.BlockSpec((1, _N, _C), lambda i: (0, 0, 0)),
            pl.BlockSpec((1, _N, _C), lambda i: (1, 0, 0)),
        ],
        out_specs=pl.BlockSpec((_N, _C), lambda i: (0, 0)),
        out_shape=jax.ShapeDtypeStruct((_N, _C), jnp.float32),
    )(q, q)


def kernel(x, edge_index, edge_weight, W1, W2):
    edges = edge_index.reshape(2, _NW, _NCHUNK, _K)
    ew = edge_weight.reshape(_NW, _EPT)

    m1 = _mm1(x, W1)
    p = _spmm_h(m1, edges, ew)
    m2 = _fuse2(p, W2)
    q = _spmm_c(m2, edges, ew)
    return _final_add(q)


# speedup vs baseline: 22.9920x; 1.1865x over previous
import functools

import jax
import jax.numpy as jnp
from jax import lax
from jax.experimental import pallas as pl
from jax.experimental.pallas import tpu as pltpu
from jax.experimental.pallas import tpu_sc as plsc

_N = 10000
_E = 320000
_D = 128
_H = 64
_C = 16

_NT = 16
_NW = 32
_K = 80
_SPB = 5
_CE = _SPB * _K
_NCHUNK = 25
_NIDX = _NCHUNK * _SPB
_EPT = _NCHUNK * _CE
_NBUF = 4
_PF = 2
_ZROWS = 125


def _make_spmm(hdim, dtype):
    mesh = plsc.VectorSubcoreMesh(core_axis_name="c", subcore_axis_name="s")
    lanes = 32 if dtype == jnp.bfloat16 else 16

    @functools.partial(
        pl.kernel,
        mesh=mesh,
        compiler_params=pltpu.CompilerParams(use_tc_tiling_on_sc=False,
                                             needs_layout_passes=False),
        out_type=jax.ShapeDtypeStruct((2, _N, hdim), dtype),
        scratch_types=[
            pltpu.VMEM((_NIDX, _K), jnp.int32),
            pltpu.VMEM((_NIDX, _K), jnp.int32),
            pltpu.VMEM((_EPT,), jnp.float32),
            pltpu.VMEM((_ZROWS, hdim), dtype),
            pltpu.VMEM_SHARED((_N, hdim), dtype),
        ] + [pltpu.VMEM((_CE, hdim), dtype) for _ in range(_NBUF)]
          + [pltpu.SemaphoreType.DMA for _ in range(2 * _NBUF)],
    )
    def spmm(m_hbm, edge_hbm, ew_hbm, out_hbm,
             src_v, dst_v, ew_v, zero_v, acc_sh,
             rb0, rb1, rb2, rb3,
             gs0, gs1, gs2, gs3, ss0, ss1, ss2, ss3):
        rbufs = (rb0, rb1, rb2, rb3)
        gsems = (gs0, gs1, gs2, gs3)
        ssems = (ss0, ss1, ss2, ss3)

        cid = lax.axis_index("c")
        sid = lax.axis_index("s")
        wid = cid * _NT + sid

        pltpu.sync_copy(edge_hbm.at[0, wid], src_v)
        pltpu.sync_copy(edge_hbm.at[1, wid], dst_v)
        pltpu.sync_copy(ew_hbm.at[wid], ew_v)

        def zero_row(r, carry):
            for j in range(hdim // lanes):
                zero_v[r, pl.ds(j * lanes, lanes)] = jnp.zeros((lanes,), dtype)
            return carry
        lax.fori_loop(0, _ZROWS, zero_row, 0)

        def zero_copy(q, carry):
            pltpu.sync_copy(
                zero_v,
                acc_sh.at[pl.ds(sid * (_N // _NT) + q * _ZROWS, _ZROWS)])
            return carry
        lax.fori_loop(0, (_N // _NT) // _ZROWS, zero_copy, 0)
        plsc.subcore_barrier()

        def start_gather(b, g):
            for s in range(_SPB):
                pltpu.async_copy(m_hbm.at[src_v.at[g * _SPB + s]],
                                 rbufs[b].at[pl.ds(s * _K, _K)], gsems[b])

        def wait_gather(b):
            for s in range(_SPB):
                pltpu.make_async_copy(m_hbm.at[src_v.at[0]],
                                      rbufs[b].at[pl.ds(s * _K, _K)],
                                      gsems[b]).wait()

        def start_scatter(b, g):
            for s in range(_SPB):
                pltpu.async_copy(rbufs[b].at[pl.ds(s * _K, _K)],
                                 acc_sh.at[dst_v.at[g * _SPB + s]], ssems[b],
                                 add=True)

        def wait_scatter(b):
            for s in range(_SPB):
                pltpu.make_async_copy(rbufs[b].at[pl.ds(s * _K, _K)],
                                      acc_sh.at[dst_v.at[0]],
                                      ssems[b]).wait()

        def scale(b, g):
            def grp(eg, carry):
                ew16 = ew_v[pl.ds(g * _CE + eg * 16, 16)]
                base = eg * 16
                for i in range(16):
                    wv = lax.broadcast(ew16[i], (16,))
                    if dtype == jnp.bfloat16:
                        wv = plsc.pack(wv, wv,
                                       format=plsc.PackFormat.INTERLEAVED)
                    for j in range(hdim // lanes):
                        rbufs[b][base + i, pl.ds(j * lanes, lanes)] = (
                            rbufs[b][base + i, pl.ds(j * lanes, lanes)] * wv)
                return carry
            lax.fori_loop(0, _CE // 16, grp, 0)

        def step(b, g, pre_b, pre_g, do_wait_scatter, do_prefetch):
            if do_prefetch:
                if do_wait_scatter:
                    wait_scatter(pre_b)
                start_gather(pre_b, pre_g)
            wait_gather(b)
            scale(b, g)
            start_scatter(b, g)

        start_gather(0, 0)
        start_gather(1, 1)
        for b in range(_NBUF):
            g = b
            step(b, g, (b + _PF) % _NBUF, g + _PF,
                 do_wait_scatter=(b >= _PF), do_prefetch=True)

        def body4(i, carry):
            for b in range(_NBUF):
                g = i * _NBUF + b
                step(b, g, (b + _PF) % _NBUF, g + _PF,
                     do_wait_scatter=True, do_prefetch=True)
            return carry
        lax.fori_loop(1, (_NCHUNK - _SPB) // _NBUF, body4, 0)

        for b in range(_NBUF):
            g = _NCHUNK - 5 + b
            pf = g + _PF < _NCHUNK
            step(b, g, (b + _PF) % _NBUF, g + _PF,
                 do_wait_scatter=pf, do_prefetch=pf)
        step(0, _NCHUNK - 1, 0, 0, do_wait_scatter=False, do_prefetch=False)
        for b in range(_NBUF):
            wait_scatter(b)
        plsc.subcore_barrier()

        @pl.when(sid < 10)
        def _():
            pltpu.sync_copy(acc_sh.at[pl.ds(sid * 1000, 1000)],
                            out_hbm.at[cid, pl.ds(sid * 1000, 1000)])

    return spmm


_spmm_h = _make_spmm(_H, jnp.bfloat16)
_spmm_c = _make_spmm(_C, jnp.float32)


def _mm_body(x_ref, w_ref, o_ref):
    o_ref[...] = jnp.dot(x_ref[...], w_ref[...],
                         preferred_element_type=jnp.float32
                         ).astype(jnp.bfloat16)


def _fuse_body(p0_ref, p1_ref, w_ref, o_ref):
    h = jnp.maximum(p0_ref[0].astype(jnp.float32)
                    + p1_ref[0].astype(jnp.float32), 0.0)
    o_ref[...] = jnp.dot(h, w_ref[...], preferred_element_type=jnp.float32)


def _add_body(a_ref, b_ref, o_ref):
    o_ref[...] = a_ref[0] + b_ref[0]


_RB = 2000


def _mm1(x, w):
    return pl.pallas_call(
        _mm_body,
        grid=(_N // _RB,),
        in_specs=[
            pl.BlockSpec((_RB, _D), lambda i: (i, 0)),
            pl.BlockSpec((_D, _H), lambda i: (0, 0)),
        ],
        out_specs=pl.BlockSpec((_RB, _H), lambda i: (i, 0)),
        out_shape=jax.ShapeDtypeStruct((_N, _H), jnp.bfloat16),
    )(x, w)


def _fuse2(p, w):
    return pl.pallas_call(
        _fuse_body,
        grid=(_N // _RB,),
        in_specs=[
            pl.BlockSpec((1, _RB, _H), lambda i: (0, i, 0)),
            pl.BlockSpec((1, _RB, _H), lambda i: (1, i, 0)),
            pl.BlockSpec((_H, _C), lambda i: (0, 0)),
        ],
        out_specs=pl.BlockSpec((_RB, _C), lambda i: (i, 0)),
        out_shape=jax.ShapeDtypeStruct((_N, _C), jnp.float32),
    )(p, p, w)


def _final_add(q):
    return pl.pallas_call(
        _add_body,
        grid=(1,),
        in_specs=[
            pl.BlockSpec((1, _N, _C), lambda i: (0, 0, 0)),
            pl.BlockSpec((1, _N, _C), lambda i: (1, 0, 0)),
        ],
        out_specs=pl.BlockSpec((_N, _C), lambda i: (0, 0)),
        out_shape=jax.ShapeDtypeStruct((_N, _C), jnp.float32),
    )(q, q)


def kernel(x, edge_index, edge_weight, W1, W2):
    edges = edge_index.reshape(2, _NW, _NIDX, _K)
    ew = edge_weight.reshape(_NW, _EPT)

    m1 = _mm1(x, W1)
    p = _spmm_h(m1, edges, ew)
    m2 = _fuse2(p, W2)
    q = _spmm_c(m2, edges, ew)
    return _final_add(q)


# speedup vs baseline: 23.8039x; 1.0353x over previous
import functools

import jax
import jax.numpy as jnp
from jax import lax
from jax.experimental import pallas as pl
from jax.experimental.pallas import tpu as pltpu
from jax.experimental.pallas import tpu_sc as plsc

_N = 10000
_E = 320000
_D = 128
_H = 64
_C = 16

_NT = 16
_NW = 32
_K = 80
_SPB = 5
_CE = _SPB * _K
_NCHUNK = 25
_NIDX = _NCHUNK * _SPB
_EPT = _NCHUNK * _CE
_NBUF = 4
_PF = 2
_ZROWS = 125


def _make_spmm(hdim, dtype):
    mesh = plsc.VectorSubcoreMesh(core_axis_name="c", subcore_axis_name="s")
    lanes = 32 if dtype == jnp.bfloat16 else 16

    @functools.partial(
        pl.kernel,
        mesh=mesh,
        compiler_params=pltpu.CompilerParams(use_tc_tiling_on_sc=False,
                                             needs_layout_passes=False),
        out_type=jax.ShapeDtypeStruct((2, _N, hdim), dtype),
        scratch_types=[
            pltpu.VMEM((_NIDX, _K), jnp.int32),
            pltpu.VMEM((_NIDX, _K), jnp.int32),
            pltpu.VMEM((_EPT,), jnp.float32),
            pltpu.VMEM((_ZROWS, hdim), dtype),
            pltpu.VMEM_SHARED((_N, hdim), dtype),
        ] + [pltpu.VMEM((_CE, hdim), dtype) for _ in range(_NBUF)]
          + [pltpu.SemaphoreType.DMA for _ in range(2 * _NBUF)],
    )
    def spmm(m_hbm, edge_hbm, ew_hbm, out_hbm,
             src_v, dst_v, ew_v, zero_v, acc_sh,
             rb0, rb1, rb2, rb3,
             gs0, gs1, gs2, gs3, ss0, ss1, ss2, ss3):
        rbufs = (rb0, rb1, rb2, rb3)
        gsems = (gs0, gs1, gs2, gs3)
        ssems = (ss0, ss1, ss2, ss3)

        cid = lax.axis_index("c")
        sid = lax.axis_index("s")
        wid = cid * _NT + sid

        cp_src = pltpu.async_copy(edge_hbm.at[0, wid], src_v, gs0)
        cp_dst = pltpu.async_copy(edge_hbm.at[1, wid], dst_v, gs1)
        cp_ew = pltpu.async_copy(ew_hbm.at[wid], ew_v, gs2)

        def zero_row(r, carry):
            for j in range(hdim // lanes):
                zero_v[r, pl.ds(j * lanes, lanes)] = jnp.zeros((lanes,), dtype)
            return carry
        lax.fori_loop(0, _ZROWS, zero_row, 0)

        def zero_copy(q, carry):
            pltpu.sync_copy(
                zero_v,
                acc_sh.at[pl.ds(sid * (_N // _NT) + q * _ZROWS, _ZROWS)])
            return carry
        lax.fori_loop(0, (_N // _NT) // _ZROWS, zero_copy, 0)
        cp_src.wait()
        cp_dst.wait()
        cp_ew.wait()
        plsc.subcore_barrier()

        def start_gather(b, g):
            for s in range(_SPB):
                pltpu.async_copy(m_hbm.at[src_v.at[g * _SPB + s]],
                                 rbufs[b].at[pl.ds(s * _K, _K)], gsems[b])

        def wait_gather(b):
            for s in range(_SPB):
                pltpu.make_async_copy(m_hbm.at[src_v.at[0]],
                                      rbufs[b].at[pl.ds(s * _K, _K)],
                                      gsems[b]).wait()

        def start_scatter(b, g):
            for s in range(_SPB):
                pltpu.async_copy(rbufs[b].at[pl.ds(s * _K, _K)],
                                 acc_sh.at[dst_v.at[g * _SPB + s]], ssems[b],
                                 add=True)

        def wait_scatter(b):
            for s in range(_SPB):
                pltpu.make_async_copy(rbufs[b].at[pl.ds(s * _K, _K)],
                                      acc_sh.at[dst_v.at[0]],
                                      ssems[b]).wait()

        def scale(b, g):
            def grp(eg, carry):
                ew16 = ew_v[pl.ds(g * _CE + eg * 16, 16)]
                base = eg * 16
                for i in range(16):
                    wv = lax.broadcast(ew16[i], (16,))
                    if dtype == jnp.bfloat16:
                        wv = plsc.pack(wv, wv,
                                       format=plsc.PackFormat.INTERLEAVED)
                    for j in range(hdim // lanes):
                        rbufs[b][base + i, pl.ds(j * lanes, lanes)] = (
                            rbufs[b][base + i, pl.ds(j * lanes, lanes)] * wv)
                return carry
            lax.fori_loop(0, _CE // 16, grp, 0)

        def step(b, g, pre_b, pre_g, do_wait_scatter, do_prefetch):
            if do_prefetch:
                if do_wait_scatter:
                    wait_scatter(pre_b)
                start_gather(pre_b, pre_g)
            wait_gather(b)
            scale(b, g)
            start_scatter(b, g)

        start_gather(0, 0)
        start_gather(1, 1)
        for b in range(_NBUF):
            g = b
            step(b, g, (b + _PF) % _NBUF, g + _PF,
                 do_wait_scatter=(b >= _PF), do_prefetch=True)

        def body4(i, carry):
            for b in range(_NBUF):
                g = i * _NBUF + b
                step(b, g, (b + _PF) % _NBUF, g + _PF,
                     do_wait_scatter=True, do_prefetch=True)
            return carry
        lax.fori_loop(1, (_NCHUNK - _SPB) // _NBUF, body4, 0)

        for b in range(_NBUF):
            g = _NCHUNK - 5 + b
            pf = g + _PF < _NCHUNK
            step(b, g, (b + _PF) % _NBUF, g + _PF,
                 do_wait_scatter=pf, do_prefetch=pf)
        step(0, _NCHUNK - 1, 0, 0, do_wait_scatter=False, do_prefetch=False)
        for b in range(_NBUF):
            wait_scatter(b)
        plsc.subcore_barrier()

        pltpu.sync_copy(acc_sh.at[pl.ds(sid * 625, 625)],
                        out_hbm.at[cid, pl.ds(sid * 625, 625)])

    return spmm


_spmm_h = _make_spmm(_H, jnp.bfloat16)
_spmm_c = _make_spmm(_C, jnp.float32)


def _mm_body(x_ref, w_ref, o_ref):
    o_ref[...] = jnp.dot(x_ref[...], w_ref[...],
                         preferred_element_type=jnp.float32
                         ).astype(jnp.bfloat16)


def _fuse_body(p0_ref, p1_ref, w_ref, o_ref):
    h = jnp.maximum(p0_ref[0].astype(jnp.float32)
                    + p1_ref[0].astype(jnp.float32), 0.0)
    o_ref[...] = jnp.dot(h, w_ref[...], preferred_element_type=jnp.float32)


def _add_body(a_ref, b_ref, o_ref):
    o_ref[...] = a_ref[0] + b_ref[0]


_RB = 2000


def _mm1(x, w):
    return pl.pallas_call(
        _mm_body,
        grid=(_N // _RB,),
        in_specs=[
            pl.BlockSpec((_RB, _D), lambda i: (i, 0)),
            pl.BlockSpec((_D, _H), lambda i: (0, 0)),
        ],
        out_specs=pl.BlockSpec((_RB, _H), lambda i: (i, 0)),
        out_shape=jax.ShapeDtypeStruct((_N, _H), jnp.bfloat16),
    )(x, w)


def _fuse2(p, w):
    return pl.pallas_call(
        _fuse_body,
        grid=(_N // _RB,),
        in_specs=[
            pl.BlockSpec((1, _RB, _H), lambda i: (0, i, 0)),
            pl.BlockSpec((1, _RB, _H), lambda i: (1, i, 0)),
            pl.BlockSpec((_H, _C), lambda i: (0, 0)),
        ],
        out_specs=pl.BlockSpec((_RB, _C), lambda i: (i, 0)),
        out_shape=jax.ShapeDtypeStruct((_N, _C), jnp.float32),
    )(p, p, w)


def _final_add(q):
    return pl.pallas_call(
        _add_body,
        grid=(1,),
        in_specs=[
            pl.BlockSpec((1, _N, _C), lambda i: (0, 0, 0)),
            pl.BlockSpec((1, _N, _C), lambda i: (1, 0, 0)),
        ],
        out_specs=pl.BlockSpec((_N, _C), lambda i: (0, 0)),
        out_shape=jax.ShapeDtypeStruct((_N, _C), jnp.float32),
    )(q, q)


def kernel(x, edge_index, edge_weight, W1, W2):
    edges = edge_index.reshape(2, _NW, _NIDX, _K)
    ew = edge_weight.reshape(_NW, _EPT)

    m1 = _mm1(x, W1)
    p = _spmm_h(m1, edges, ew)
    m2 = _fuse2(p, W2)
    q = _spmm_c(m2, edges, ew)
    return _final_add(q)
